# baseline (speedup 1.0000x reference)
import jax
import jax.numpy as jnp
from jax.experimental import pallas as pl
from jax.experimental.pallas import tpu as pltpu

_T = 32
_LANES = 128


def _mod_kernel(ls_ref, x_ref, o_ref):
    ls = ls_ref[0, 0]
    timescale = jnp.exp(ls * 100.0)
    rows = jax.lax.broadcasted_iota(jnp.int32, (_LANES, _LANES), 0)
    cols = jax.lax.broadcasted_iota(jnp.int32, (_LANES, _LANES), 1)
    tc = (cols % _T).astype(jnp.float32)
    t_idx = jnp.clip(tc / timescale, 0.0, float(_T - 1))
    lower = jnp.floor(t_idx).astype(jnp.int32)
    upper = jnp.minimum(lower + 1, _T - 1)
    w = t_idx - lower.astype(jnp.float32)
    ts = rows % _T
    same = (rows // _T) == (cols // _T)
    m = jnp.where(same & (ts == lower), 1.0 - w, 0.0) + jnp.where(
        same & (ts == upper), w, 0.0
    )
    o_ref[...] = jnp.dot(x_ref[...], m, preferred_element_type=jnp.float32)


def kernel(x, log_timescale):
    B, H, W, T = x.shape
    n = (B * H * W * T) // _LANES
    x2d = x.reshape(n, _LANES)
    blk = 2048
    grid = n // blk
    out = pl.pallas_call(
        _mod_kernel,
        grid=(grid,),
        in_specs=[
            pl.BlockSpec(memory_space=pltpu.SMEM),
            pl.BlockSpec((blk, _LANES), lambda i: (i, 0)),
        ],
        out_specs=pl.BlockSpec((blk, _LANES), lambda i: (i, 0)),
        out_shape=jax.ShapeDtypeStruct((n, _LANES), jnp.float32),
        compiler_params=pltpu.CompilerParams(
            dimension_semantics=("arbitrary",),
        ),
    )(log_timescale.reshape(1, 1), x2d)
    return out.reshape(B, H, W, T)


# speedup vs baseline: 10.4864x; 10.4864x over previous
import jax
import jax.numpy as jnp
from jax.experimental import pallas as pl
from jax.experimental.pallas import tpu as pltpu

_T = 32
_R = 64


def _mod_kernel(ls_ref, x_ref, o_ref):
    ls = ls_ref[0, 0]
    timescale = jnp.exp(ls * 100.0)
    trow = jax.lax.broadcasted_iota(jnp.int32, (_T, _T), 0)
    scol = jax.lax.broadcasted_iota(jnp.int32, (_T, _T), 1)
    t_idx = jnp.clip(trow.astype(jnp.float32) / timescale, 0.0, float(_T - 1))
    lower = jnp.floor(t_idx).astype(jnp.int32)
    upper = jnp.minimum(lower + 1, _T - 1)
    w = t_idx - lower.astype(jnp.float32)
    m = jnp.where(scol == lower, 1.0 - w, 0.0) + jnp.where(scol == upper, w, 0.0)
    for r in range(_R):
        o_ref[r] = jnp.dot(m, x_ref[r], preferred_element_type=jnp.float32)


def kernel(x, log_timescale):
    B, H, W, T = x.shape
    g = B * H
    xt = jnp.transpose(x, (0, 1, 3, 2)).reshape(g, T, W)
    grid = g // _R
    out = pl.pallas_call(
        _mod_kernel,
        grid=(grid,),
        in_specs=[
            pl.BlockSpec(memory_space=pltpu.SMEM),
            pl.BlockSpec((_R, T, W), lambda i: (i, 0, 0)),
        ],
        out_specs=pl.BlockSpec((_R, T, W), lambda i: (i, 0, 0)),
        out_shape=jax.ShapeDtypeStruct((g, T, W), jnp.float32),
        compiler_params=pltpu.CompilerParams(
            dimension_semantics=("arbitrary",),
        ),
    )(log_timescale.reshape(1, 1), xt)
    return jnp.transpose(out.reshape(B, H, T, W), (0, 1, 3, 2))


# speedup vs baseline: 11.9185x; 1.1366x over previous
import jax
import jax.numpy as jnp
from jax.experimental import pallas as pl
from jax.experimental.pallas import tpu as pltpu

_T = 32
_R = 128


def _mod_kernel(ls_ref, x_ref, o_ref):
    ls = ls_ref[0, 0]
    timescale = jnp.exp(ls * 100.0)
    trow = jax.lax.broadcasted_iota(jnp.int32, (_T, _T), 0)
    scol = jax.lax.broadcasted_iota(jnp.int32, (_T, _T), 1)
    t_idx = jnp.clip(trow.astype(jnp.float32) / timescale, 0.0, float(_T - 1))
    lower = jnp.floor(t_idx).astype(jnp.int32)
    upper = jnp.minimum(lower + 1, _T - 1)
    w = t_idx - lower.astype(jnp.float32)
    m = jnp.where(scol == lower, 1.0 - w, 0.0) + jnp.where(scol == upper, w, 0.0)
    for r in range(_R):
        o_ref[r] = jnp.dot(m, x_ref[r], preferred_element_type=jnp.float32)


def kernel(x, log_timescale):
    B, H, W, T = x.shape
    g = B * H
    xt = jnp.transpose(x, (0, 1, 3, 2)).reshape(g, T, W)
    grid = g // _R
    out = pl.pallas_call(
        _mod_kernel,
        grid=(grid,),
        in_specs=[
            pl.BlockSpec(memory_space=pltpu.SMEM),
            pl.BlockSpec((_R, T, W), lambda i: (i, 0, 0)),
        ],
        out_specs=pl.BlockSpec((_R, T, W), lambda i: (i, 0, 0)),
        out_shape=jax.ShapeDtypeStruct((g, T, W), jnp.float32),
        compiler_params=pltpu.CompilerParams(
            dimension_semantics=("arbitrary",),
        ),
    )(log_timescale.reshape(1, 1), xt)
    return jnp.transpose(out.reshape(B, H, T, W), (0, 1, 3, 2))


# speedup vs baseline: 12.3884x; 1.0394x over previous
import jax
import jax.numpy as jnp
from jax.experimental import pallas as pl
from jax.experimental.pallas import tpu as pltpu

_T = 32
_R = 256


def _mod_kernel(ls_ref, x_ref, o_ref):
    ls = ls_ref[0, 0]
    timescale = jnp.exp(ls * 100.0)
    trow = jax.lax.broadcasted_iota(jnp.int32, (_T, _T), 0)
    scol = jax.lax.broadcasted_iota(jnp.int32, (_T, _T), 1)
    t_idx = jnp.clip(trow.astype(jnp.float32) / timescale, 0.0, float(_T - 1))
    lower = jnp.floor(t_idx).astype(jnp.int32)
    upper = jnp.minimum(lower + 1, _T - 1)
    w = t_idx - lower.astype(jnp.float32)
    m = jnp.where(scol == lower, 1.0 - w, 0.0) + jnp.where(scol == upper, w, 0.0)
    for r in range(_R):
        o_ref[r] = jnp.dot(m, x_ref[r], preferred_element_type=jnp.float32)


def kernel(x, log_timescale):
    B, H, W, T = x.shape
    g = B * H
    xt = jnp.transpose(x, (0, 1, 3, 2)).reshape(g, T, W)
    grid = g // _R
    out = pl.pallas_call(
        _mod_kernel,
        grid=(grid,),
        in_specs=[
            pl.BlockSpec(memory_space=pltpu.SMEM),
            pl.BlockSpec((_R, T, W), lambda i: (i, 0, 0)),
        ],
        out_specs=pl.BlockSpec((_R, T, W), lambda i: (i, 0, 0)),
        out_shape=jax.ShapeDtypeStruct((g, T, W), jnp.float32),
        compiler_params=pltpu.CompilerParams(
            dimension_semantics=("arbitrary",),
        ),
    )(log_timescale.reshape(1, 1), xt)
    return jnp.transpose(out.reshape(B, H, T, W), (0, 1, 3, 2))
